# Initial kernel scaffold; baseline (speedup 1.0000x reference)
import jax
import jax.numpy as jnp
from jax.experimental import pallas as pl


def kernel(indices, table):
    raise NotImplementedError("write your pallas kernel here")


# speedup vs baseline: 1.5610x; 1.5610x over previous
import functools

import jax
import jax.numpy as jnp
from jax import lax
from jax.experimental import pallas as pl
from jax.experimental.pallas import tpu as pltpu
from jax.experimental.pallas import tpu_sc as plsc

BATCH = 16384
N_FIELDS = 26
EMBED_DIM = 32

_B = BATCH * N_FIELDS
_NC, _NS = 2, 16
_NW = _NC * _NS
_B_PER_W = _B // _NW
_C = 1664
_NCHUNK = _B_PER_W // _C

_mesh = plsc.VectorSubcoreMesh(core_axis_name="c", subcore_axis_name="s")


@functools.partial(
    pl.kernel,
    mesh=_mesh,
    out_type=jax.ShapeDtypeStruct((_B, EMBED_DIM), jnp.float32),
    scratch_types=[
        pltpu.VMEM((_C,), jnp.int32),
        pltpu.VMEM((_C, EMBED_DIM), jnp.float32),
        pltpu.SemaphoreType.DMA,
    ],
    compiler_params=pltpu.CompilerParams(use_tc_tiling_on_sc=False),
)
def _sc_gather(idx_hbm, table_hbm, out_hbm, idx_v, rows_v, sem):
    wid = lax.axis_index("s") * _NC + lax.axis_index("c")
    base = wid * _B_PER_W
    for k in range(_NCHUNK):
        off = base + k * _C
        pltpu.sync_copy(idx_hbm.at[pl.ds(off, _C)], idx_v)
        pltpu.async_copy(table_hbm.at[idx_v], rows_v, sem).wait()
        pltpu.sync_copy(rows_v, out_hbm.at[pl.ds(off, _C)])


def kernel(indices, table):
    idx_flat = indices.reshape(_B).astype(jnp.int32)
    out = _sc_gather(idx_flat, table)
    return out.reshape(BATCH, N_FIELDS, EMBED_DIM)
